# baseline (speedup 1.0000x reference)
import jax
import jax.numpy as jnp
from jax.experimental import pallas as pl
from jax.experimental.pallas import tpu as pltpu


def _net_kernel(xs_ref, w1_ref, b1_ref, w2_ref, b2_ref, w1r_ref, fb1_ref,
                wf2_ref, fb2_ref, wf3_ref, fb3_ref, o_ref):
    tb = o_ref.shape[0]

    yc = []
    for j in range(4):
        x1 = jnp.concatenate(
            [xs_ref[(j + k) // 4:(j + k) // 4 + 7, :,
                    ci * 128 + ((j + k) % 4) * 32:
                    ci * 128 + ((j + k) % 4) * 32 + 32].reshape(7 * tb, 32)
             for k in range(5) for ci in range(3)], axis=-1)
        acc = jnp.dot(x1, w1_ref[...], preferred_element_type=jnp.float32)
        y = acc.reshape(7, tb, 256)
        yc.append(jnp.maximum(y[:, :, :128], y[:, :, 128:]))
    y1 = jnp.stack([jnp.maximum(yc[0], yc[1]),
                    jnp.maximum(yc[2], yc[3])], axis=1).reshape(14, tb, 128)
    y1 = jnp.maximum(y1 + b1_ref[...], 0.0)

    y1b = y1.astype(jnp.bfloat16)
    x2 = jnp.concatenate(
        [y1b[k:k + 10, :, :96].reshape(10 * tb, 96) for k in range(5)],
        axis=-1)
    acc2 = jnp.dot(x2, w2_ref[...], preferred_element_type=jnp.float32)
    z = acc2.reshape(10, tb, 256)
    z = jnp.maximum(z[:, :, :128], z[:, :, 128:])
    z = z.reshape(5, 2, tb, 128)
    z = jnp.maximum(z[:, 0], z[:, 1])
    y2 = jnp.maximum(z + b2_ref[...], 0.0)

    y2b = y2.astype(jnp.bfloat16)
    xf = jnp.concatenate([y2b[p] for p in range(5)], axis=-1)
    h1 = jnp.dot(xf, w1r_ref[...], preferred_element_type=jnp.float32)
    h1 = jnp.maximum(h1 + fb1_ref[...], 0.0).astype(jnp.bfloat16)
    h2 = jnp.maximum(
        jnp.dot(h1, wf2_ref[...], preferred_element_type=jnp.float32)
        + fb2_ref[...], 0.0).astype(jnp.bfloat16)
    o_ref[...] = (jnp.dot(h2, wf3_ref[...], preferred_element_type=jnp.float32)
                  + fb3_ref[...])[:, :10]


def _prep(w1, b1, w2, b2, fw1, fb1, fw2, fb2, fw3, fb3):
    f32 = jnp.float32
    bf16 = jnp.bfloat16
    par = jnp.arange(2)
    kj = jnp.arange(5)

    c = jnp.arange(32)
    ojp = jnp.arange(14)
    m1 = (c[:, None, None, None]
          == (2 * ojp[None, :, None, None] + par[None, None, :, None]
              + kj[None, None, None, :])).astype(f32)
    w1t = jnp.einsum('cjpq,oikq->kicpjo', m1, w1.astype(f32))
    w1t = jnp.pad(w1t.reshape(5, 96, 2, 84), ((0, 0), (0, 0), (0, 0), (0, 44)))
    w1g = w1t.reshape(480, 256)

    f = jnp.arange(14)
    oj2p = jnp.arange(5)
    m2 = (f[:, None, None, None]
          == (2 * oj2p[None, :, None, None] + par[None, None, :, None]
              + kj[None, None, None, :])).astype(f32)
    w2t = jnp.einsum('fjpq,oikq->kfipjo', m2, w2.astype(f32))
    w2t = jnp.pad(w2t.reshape(5, 84, 2, 80), ((0, 0), (0, 0), (0, 0), (0, 48)))
    w2g = jnp.pad(w2t.reshape(5, 84, 256),
                  ((0, 0), (0, 12), (0, 0))).reshape(480, 256)

    w1r = jnp.transpose(fw1.astype(f32).reshape(120, 16, 5, 5), (2, 3, 1, 0))
    w1r = jnp.pad(w1r.reshape(5, 80, 120),
                  ((0, 0), (0, 48), (0, 8))).reshape(640, 128)

    b1g = jnp.pad(jnp.tile(b1.astype(f32), 14), (0, 44)).reshape(1, 128)
    b2g = jnp.pad(jnp.tile(b2.astype(f32), 5), (0, 48)).reshape(1, 128)
    fb1g = jnp.pad(fb1.astype(f32), (0, 8)).reshape(1, 128)
    fb2g = jnp.pad(fb2.astype(f32), (0, 44)).reshape(1, 128)
    fb3g = jnp.pad(fb3.astype(f32), (0, 118)).reshape(1, 128)
    wf2g = jnp.pad(fw2.astype(f32).T, ((0, 8), (0, 44)))
    wf3g = jnp.pad(fw3.astype(f32).T, ((0, 44), (0, 118)))
    return (w1g.astype(bf16), b1g, w2g.astype(bf16), b2g, w1r.astype(bf16),
            fb1g, wf2g.astype(bf16), fb2g, wf3g.astype(bf16), fb3g)


def kernel(x, w1, b1, w2, b2, fw1, fb1, fw2, fb2, fw3, fb3):
    b = x.shape[0]
    tb = 1024
    b_pad = tb * (-(-b // tb))

    xs = jnp.transpose(x.astype(jnp.bfloat16).reshape(b, 3, 8, 128),
                       (2, 0, 1, 3)).reshape(8, b, 384)
    if b_pad != b:
        xs = jnp.pad(xs, ((0, 0), (0, b_pad - b), (0, 0)))

    pp = _prep(w1, b1, w2, b2, fw1, fb1, fw2, fb2, fw3, fb3)

    res = lambda shape: pl.BlockSpec(shape, lambda i, _s=(0,) * len(shape): _s)
    out = pl.pallas_call(
        _net_kernel,
        out_shape=jax.ShapeDtypeStruct((b_pad, 10), jnp.float32),
        grid=(b_pad // tb,),
        in_specs=[
            pl.BlockSpec((8, tb, 384), lambda i: (0, i, 0)),
            res((480, 256)), res((1, 128)),
            res((480, 256)), res((1, 128)),
            res((640, 128)), res((1, 128)),
            res((128, 128)), res((1, 128)),
            res((128, 128)), res((1, 128)),
        ],
        out_specs=pl.BlockSpec((tb, 10), lambda i: (i, 0)),
        compiler_params=pltpu.CompilerParams(
            dimension_semantics=("parallel",),
            vmem_limit_bytes=64 * 1024 * 1024),
    )(xs, *pp)
    return out[:b]


# speedup vs baseline: 1.1484x; 1.1484x over previous
import jax
import jax.numpy as jnp
from jax.experimental import pallas as pl
from jax.experimental.pallas import tpu as pltpu


def _net_kernel(xs_ref, w1_ref, b1_ref, w2_ref, b2_ref, w1r_ref, fb1_ref,
                wf2_ref, fb2_ref, wf3_ref, fb3_ref, o_ref):
    tb = o_ref.shape[0]
    half = tb

    for h in range(1):
        xs = xs_ref[:, h * half:(h + 1) * half, :]

        x1 = jnp.concatenate(
            [xs[k:k + 28].reshape(28 * half, 96) for k in range(5)], axis=-1)
        acc = jnp.dot(x1, w1_ref[...], preferred_element_type=jnp.float32)
        y = acc.reshape(28, half, 256)
        y = jnp.maximum(y[:, :, :128], y[:, :, 128:])
        y = y.reshape(14, 2, half, 128)
        y = jnp.maximum(y[:, 0], y[:, 1])
        y1 = jnp.maximum(y + b1_ref[...], 0.0)

        y1b = y1.astype(jnp.bfloat16)
        x2 = jnp.concatenate(
            [y1b[k:k + 10, :, :96].reshape(10 * half, 96) for k in range(5)],
            axis=-1)
        acc2 = jnp.dot(x2, w2_ref[...], preferred_element_type=jnp.float32)
        z = acc2.reshape(10, half, 256)
        z = jnp.maximum(z[:, :, :128], z[:, :, 128:])
        z = z.reshape(5, 2, half, 128)
        z = jnp.maximum(z[:, 0], z[:, 1])
        y2 = jnp.maximum(z + b2_ref[...], 0.0)

        y2b = y2.astype(jnp.bfloat16)
        xf = jnp.concatenate([y2b[p] for p in range(5)], axis=-1)
        h1 = jnp.dot(xf, w1r_ref[...], preferred_element_type=jnp.float32)
        h1 = jnp.maximum(h1 + fb1_ref[...], 0.0).astype(jnp.bfloat16)
        h2 = jnp.maximum(
            jnp.dot(h1, wf2_ref[...], preferred_element_type=jnp.float32)
            + fb2_ref[...], 0.0).astype(jnp.bfloat16)
        o_ref[h * half:(h + 1) * half, :] = (
            jnp.dot(h2, wf3_ref[...], preferred_element_type=jnp.float32)
            + fb3_ref[...])[:, :10]


def _prep(w1, b1, w2, b2, fw1, fb1, fw2, fb2, fw3, fb3):
    f32 = jnp.float32
    bf16 = jnp.bfloat16
    par = jnp.arange(2)
    kj = jnp.arange(5)

    c = jnp.arange(32)
    ojp = jnp.arange(14)
    m1 = (c[:, None, None, None]
          == (2 * ojp[None, :, None, None] + par[None, None, :, None]
              + kj[None, None, None, :])).astype(f32)
    w1t = jnp.einsum('cjpq,oikq->kicpjo', m1, w1.astype(f32))
    w1t = jnp.pad(w1t.reshape(5, 96, 2, 84), ((0, 0), (0, 0), (0, 0), (0, 44)))
    w1g = w1t.reshape(480, 256)

    f = jnp.arange(14)
    oj2p = jnp.arange(5)
    m2 = (f[:, None, None, None]
          == (2 * oj2p[None, :, None, None] + par[None, None, :, None]
              + kj[None, None, None, :])).astype(f32)
    w2t = jnp.einsum('fjpq,oikq->kfipjo', m2, w2.astype(f32))
    w2t = jnp.pad(w2t.reshape(5, 84, 2, 80), ((0, 0), (0, 0), (0, 0), (0, 48)))
    w2g = jnp.pad(w2t.reshape(5, 84, 256),
                  ((0, 0), (0, 12), (0, 0))).reshape(480, 256)

    w1r = jnp.transpose(fw1.astype(f32).reshape(120, 16, 5, 5), (2, 3, 1, 0))
    w1r = jnp.pad(w1r.reshape(5, 80, 120),
                  ((0, 0), (0, 48), (0, 8))).reshape(640, 128)

    b1g = jnp.pad(jnp.tile(b1.astype(f32), 14), (0, 44)).reshape(1, 128)
    b2g = jnp.pad(jnp.tile(b2.astype(f32), 5), (0, 48)).reshape(1, 128)
    fb1g = jnp.pad(fb1.astype(f32), (0, 8)).reshape(1, 128)
    fb2g = jnp.pad(fb2.astype(f32), (0, 44)).reshape(1, 128)
    fb3g = jnp.pad(fb3.astype(f32), (0, 118)).reshape(1, 128)
    wf2g = jnp.pad(fw2.astype(f32).T, ((0, 8), (0, 44)))
    wf3g = jnp.pad(fw3.astype(f32).T, ((0, 44), (0, 118)))
    return (w1g.astype(bf16), b1g, w2g.astype(bf16), b2g, w1r.astype(bf16),
            fb1g, wf2g.astype(bf16), fb2g, wf3g.astype(bf16), fb3g)


def kernel(x, w1, b1, w2, b2, fw1, fb1, fw2, fb2, fw3, fb3):
    b = x.shape[0]
    tb = 1024
    b_pad = tb * (-(-b // tb))

    xs = jnp.transpose(x.astype(jnp.bfloat16), (2, 0, 1, 3)).reshape(32, b, 96)
    if b_pad != b:
        xs = jnp.pad(xs, ((0, 0), (0, b_pad - b), (0, 0)))

    pp = _prep(w1, b1, w2, b2, fw1, fb1, fw2, fb2, fw3, fb3)

    res = lambda shape: pl.BlockSpec(shape, lambda i, _s=(0,) * len(shape): _s)
    out = pl.pallas_call(
        _net_kernel,
        out_shape=jax.ShapeDtypeStruct((b_pad, 10), jnp.float32),
        grid=(b_pad // tb,),
        in_specs=[
            pl.BlockSpec((32, tb, 96), lambda i: (0, i, 0)),
            res((480, 256)), res((1, 128)),
            res((480, 256)), res((1, 128)),
            res((640, 128)), res((1, 128)),
            res((128, 128)), res((1, 128)),
            res((128, 128)), res((1, 128)),
        ],
        out_specs=pl.BlockSpec((tb, 10), lambda i: (i, 0)),
        compiler_params=pltpu.CompilerParams(
            dimension_semantics=("parallel",),
            vmem_limit_bytes=64 * 1024 * 1024),
    )(xs, *pp)
    return out[:b]


# speedup vs baseline: 1.1502x; 1.0016x over previous
import jax
import jax.numpy as jnp
from jax.experimental import pallas as pl
from jax.experimental.pallas import tpu as pltpu


def _net_kernel(xs_ref, w1_ref, b1_ref, w2_ref, b2_ref, w1r_ref, fb1_ref,
                wf2_ref, fb2_ref, wf3_ref, fb3_ref, o_ref):
    tb = o_ref.shape[0]

    x1 = jnp.concatenate(
        [xs_ref[k:k + 28].reshape(28 * tb, 96) for k in range(5)], axis=-1)
    acc = jnp.dot(x1, w1_ref[...], preferred_element_type=jnp.float32)
    y = acc.reshape(28, tb, 256)
    y = jnp.maximum(y[:, :, :128], y[:, :, 128:])
    y = y.reshape(14, 2, tb, 128)
    y = jnp.maximum(y[:, 0], y[:, 1])
    y1 = jnp.maximum(y + b1_ref[...], 0.0)

    y1b = y1.astype(jnp.bfloat16)
    x2 = jnp.concatenate(
        [y1b[k:k + 10, :, :96].reshape(10 * tb, 96) for k in range(5)],
        axis=-1)
    acc2 = jnp.dot(x2, w2_ref[...], preferred_element_type=jnp.float32)
    z = acc2.reshape(10, tb, 256)
    z = jnp.maximum(z[:, :, :128], z[:, :, 128:])
    z = z.reshape(5, 2, tb, 128)
    z = jnp.maximum(z[:, 0], z[:, 1])
    y2 = jnp.maximum(z + b2_ref[...], 0.0)

    y2b = y2.astype(jnp.bfloat16)
    xf = jnp.concatenate([y2b[p] for p in range(5)], axis=-1)
    h1 = jnp.dot(xf, w1r_ref[...], preferred_element_type=jnp.float32)
    h1 = jnp.maximum(h1 + fb1_ref[...], 0.0).astype(jnp.bfloat16)
    h2 = jnp.maximum(
        jnp.dot(h1, wf2_ref[...], preferred_element_type=jnp.float32)
        + fb2_ref[...], 0.0).astype(jnp.bfloat16)
    o_ref[...] = (jnp.dot(h2, wf3_ref[...], preferred_element_type=jnp.float32)
                  + fb3_ref[...])[:, :10]


def _prep(w1, b1, w2, b2, fw1, fb1, fw2, fb2, fw3, fb3):
    f32 = jnp.float32
    bf16 = jnp.bfloat16
    par = jnp.arange(2)
    kj = jnp.arange(5)

    c = jnp.arange(32)
    ojp = jnp.arange(14)
    m1 = (c[:, None, None, None]
          == (2 * ojp[None, :, None, None] + par[None, None, :, None]
              + kj[None, None, None, :])).astype(f32)
    w1t = jnp.einsum('cjpq,oikq->kicpjo', m1, w1.astype(f32))
    w1t = jnp.pad(w1t.reshape(5, 96, 2, 84), ((0, 0), (0, 0), (0, 0), (0, 44)))
    w1g = w1t.reshape(480, 256)

    f = jnp.arange(14)
    oj2p = jnp.arange(5)
    m2 = (f[:, None, None, None]
          == (2 * oj2p[None, :, None, None] + par[None, None, :, None]
              + kj[None, None, None, :])).astype(f32)
    w2t = jnp.einsum('fjpq,oikq->kfipjo', m2, w2.astype(f32))
    w2t = jnp.pad(w2t.reshape(5, 84, 2, 80), ((0, 0), (0, 0), (0, 0), (0, 48)))
    w2g = jnp.pad(w2t.reshape(5, 84, 256),
                  ((0, 0), (0, 12), (0, 0))).reshape(480, 256)

    w1r = jnp.transpose(fw1.astype(f32).reshape(120, 16, 5, 5), (2, 3, 1, 0))
    w1r = jnp.pad(w1r.reshape(5, 80, 120),
                  ((0, 0), (0, 48), (0, 8))).reshape(640, 128)

    b1g = jnp.pad(jnp.tile(b1.astype(f32), 14), (0, 44)).reshape(1, 128)
    b2g = jnp.pad(jnp.tile(b2.astype(f32), 5), (0, 48)).reshape(1, 128)
    fb1g = jnp.pad(fb1.astype(f32), (0, 8)).reshape(1, 128)
    fb2g = jnp.pad(fb2.astype(f32), (0, 44)).reshape(1, 128)
    fb3g = jnp.pad(fb3.astype(f32), (0, 118)).reshape(1, 128)
    wf2g = jnp.pad(fw2.astype(f32).T, ((0, 8), (0, 44)))
    wf3g = jnp.pad(fw3.astype(f32).T, ((0, 44), (0, 118)))
    return (w1g.astype(bf16), b1g, w2g.astype(bf16), b2g, w1r.astype(bf16),
            fb1g, wf2g.astype(bf16), fb2g, wf3g.astype(bf16), fb3g)


def kernel(x, w1, b1, w2, b2, fw1, fb1, fw2, fb2, fw3, fb3):
    b = x.shape[0]
    tb = 1024
    b_pad = tb * (-(-b // tb))

    xs = jnp.transpose(x.astype(jnp.bfloat16), (2, 0, 1, 3)).reshape(32, b, 96)
    if b_pad != b:
        xs = jnp.pad(xs, ((0, 0), (0, b_pad - b), (0, 0)))

    pp = _prep(w1, b1, w2, b2, fw1, fb1, fw2, fb2, fw3, fb3)

    res = lambda shape: pl.BlockSpec(shape, lambda i, _s=(0,) * len(shape): _s)
    out = pl.pallas_call(
        _net_kernel,
        out_shape=jax.ShapeDtypeStruct((b_pad, 10), jnp.float32),
        grid=(b_pad // tb,),
        in_specs=[
            pl.BlockSpec((32, tb, 96), lambda i: (0, i, 0)),
            res((480, 256)), res((1, 128)),
            res((480, 256)), res((1, 128)),
            res((640, 128)), res((1, 128)),
            res((128, 128)), res((1, 128)),
            res((128, 128)), res((1, 128)),
        ],
        out_specs=pl.BlockSpec((tb, 10), lambda i: (i, 0)),
        compiler_params=pltpu.CompilerParams(
            dimension_semantics=("parallel",),
            vmem_limit_bytes=64 * 1024 * 1024),
    )(xs, *pp)
    return out[:b]
